# Initial kernel scaffold; baseline (speedup 1.0000x reference)
import jax
import jax.numpy as jnp
from jax.experimental import pallas as pl


def kernel(x, w_gate, W1, b1, W2, b2):
    raise NotImplementedError("write your pallas kernel here")


# speedup vs baseline: 1.7805x; 1.7805x over previous
import functools

import jax
import jax.numpy as jnp
from jax.experimental import pallas as pl
from jax.experimental.pallas import tpu as pltpu

B = 8192
D = 256
E = 16
H = 2048
O = 256

BT = 256
NI = B // BT


def _gating_kernel(x_ref, wg_ref, gates_ref, loss_ref):
    lg = jnp.dot(x_ref[...], wg_ref[...], preferred_element_type=jnp.float32)
    m = jnp.max(lg, axis=1, keepdims=True)
    ex = jnp.exp(lg - m)
    g = ex / jnp.sum(ex, axis=1, keepdims=True)
    gates_ref[...] = g
    imp = jnp.sum(g, axis=0)
    load = jnp.sum((g > 0).astype(jnp.float32), axis=0)

    def cv_sq(v):
        mu = jnp.mean(v)
        var = jnp.sum((v - mu) ** 2) / (E - 1)
        return var / (mu * mu + 1e-10)

    loss_ref[...] = jnp.broadcast_to(cv_sq(imp) + cv_sq(load), (1, 1))


def _expert_kernel(x_ref, w1_ref, b1_ref, w2_ref, b2_ref, gates_ref, y_ref):
    e = pl.program_id(0)
    i = pl.program_id(1)
    rows = pl.ds(i * BT, BT)

    xb = x_ref[rows, :]
    h = jnp.dot(xb, w1_ref[0], preferred_element_type=jnp.float32)
    h = jnp.maximum(h + b1_ref[0], 0.0)
    l2 = jnp.dot(h.astype(jnp.bfloat16), w2_ref[0],
                 preferred_element_type=jnp.float32)
    l2 = l2 + b2_ref[0]
    m2 = jnp.max(l2, axis=1, keepdims=True)
    p = jnp.exp(l2 - m2)
    p = p / jnp.sum(p, axis=1, keepdims=True)

    g_blk = gates_ref[rows, :]
    onehot = (jax.lax.broadcasted_iota(jnp.int32, (1, E), 1) == e)
    gate_col = jnp.sum(g_blk * onehot.astype(jnp.float32), axis=1,
                       keepdims=True)
    contrib = gate_col * p

    @pl.when(e == 0)
    def _():
        y_ref[rows, :] = contrib

    @pl.when(e > 0)
    def _():
        y_ref[rows, :] = y_ref[rows, :] + contrib


@functools.partial(jax.jit, static_argnames=("interpret",))
def kernel(x, w_gate, W1, b1, W2, b2, interpret=False):
    gates, loss = pl.pallas_call(
        _gating_kernel,
        out_shape=[
            jax.ShapeDtypeStruct((B, E), jnp.float32),
            jax.ShapeDtypeStruct((1, 1), jnp.float32),
        ],
        interpret=interpret,
    )(x, w_gate)

    x16 = x.astype(jnp.bfloat16)
    W1_16 = W1.astype(jnp.bfloat16)
    W2_16 = W2.astype(jnp.bfloat16)

    y = pl.pallas_call(
        _expert_kernel,
        grid=(E, NI),
        in_specs=[
            pl.BlockSpec((B, D), lambda e, i: (0, 0)),
            pl.BlockSpec((1, D, H), lambda e, i: (e, 0, 0)),
            pl.BlockSpec((1, 1, H), lambda e, i: (e, 0, 0)),
            pl.BlockSpec((1, H, O), lambda e, i: (e, 0, 0)),
            pl.BlockSpec((1, 1, O), lambda e, i: (e, 0, 0)),
            pl.BlockSpec((B, E), lambda e, i: (0, 0)),
        ],
        out_specs=pl.BlockSpec((B, O), lambda e, i: (0, 0)),
        out_shape=jax.ShapeDtypeStruct((B, O), jnp.float32),
        interpret=interpret,
    )(x16, W1_16, b1.reshape(E, 1, H), W2_16, b2.reshape(E, 1, O), gates)

    return (y, loss[0, 0])


# speedup vs baseline: 2.3784x; 1.3358x over previous
import functools

import jax
import jax.numpy as jnp
from jax.experimental import pallas as pl
from jax.experimental.pallas import tpu as pltpu

B = 8192
D = 256
E = 16
H = 2048
O = 256

BT = 512
NI = B // BT


def _gating_kernel(x_ref, wg_ref, gates_ref, loss_ref):
    lg = jnp.dot(x_ref[...], wg_ref[...], preferred_element_type=jnp.float32)
    m = jnp.max(lg, axis=1, keepdims=True)
    ex = jnp.exp(lg - m)
    g = ex / jnp.sum(ex, axis=1, keepdims=True)
    gates_ref[...] = g
    imp = jnp.sum(g, axis=0)
    load = jnp.sum((g > 0).astype(jnp.float32), axis=0)

    def cv_sq(v):
        mu = jnp.mean(v)
        var = jnp.sum((v - mu) ** 2) / (E - 1)
        return var / (mu * mu + 1e-10)

    loss_ref[...] = jnp.broadcast_to(cv_sq(imp) + cv_sq(load), (1, 1))


def _expert_kernel(x_ref, w1_ref, b1_ref, w2_ref, b2_ref, gates_ref, y_ref):
    e = pl.program_id(0)
    i = pl.program_id(1)
    rows = pl.ds(i * BT, BT)

    xb = x_ref[rows, :]
    h = jnp.dot(xb, w1_ref[0], preferred_element_type=jnp.float32)
    h = jnp.maximum((h + b1_ref[0]).astype(jnp.bfloat16), jnp.bfloat16(0))
    l2 = jnp.dot(h, w2_ref[0], preferred_element_type=jnp.float32)
    l2 = l2 + b2_ref[0]
    p = jnp.exp(l2)
    p = p / jnp.sum(p, axis=1, keepdims=True)

    g_blk = gates_ref[rows, :]
    onehot = (jax.lax.broadcasted_iota(jnp.int32, (1, E), 1) == e)
    gate_col = jnp.sum(g_blk * onehot.astype(jnp.float32), axis=1,
                       keepdims=True)
    contrib = gate_col * p

    @pl.when(e == 0)
    def _():
        y_ref[rows, :] = contrib

    @pl.when(e > 0)
    def _():
        y_ref[rows, :] = y_ref[rows, :] + contrib


@functools.partial(jax.jit, static_argnames=("interpret",))
def kernel(x, w_gate, W1, b1, W2, b2, interpret=False):
    gates, loss = pl.pallas_call(
        _gating_kernel,
        out_shape=[
            jax.ShapeDtypeStruct((B, E), jnp.float32),
            jax.ShapeDtypeStruct((1, 1), jnp.float32),
        ],
        interpret=interpret,
    )(x, w_gate)

    x16 = x.astype(jnp.bfloat16)
    W1_16 = W1.astype(jnp.bfloat16)
    W2_16 = W2.astype(jnp.bfloat16)

    y = pl.pallas_call(
        _expert_kernel,
        grid=(E, NI),
        in_specs=[
            pl.BlockSpec((B, D), lambda e, i: (0, 0)),
            pl.BlockSpec((1, D, H), lambda e, i: (e, 0, 0)),
            pl.BlockSpec((1, 1, H), lambda e, i: (e, 0, 0)),
            pl.BlockSpec((1, H, O), lambda e, i: (e, 0, 0)),
            pl.BlockSpec((1, 1, O), lambda e, i: (e, 0, 0)),
            pl.BlockSpec((B, E), lambda e, i: (0, 0)),
        ],
        out_specs=pl.BlockSpec((B, O), lambda e, i: (0, 0)),
        out_shape=jax.ShapeDtypeStruct((B, O), jnp.float32),
        interpret=interpret,
    )(x16, W1_16, b1.reshape(E, 1, H), W2_16, b2.reshape(E, 1, O), gates)

    return (y, loss[0, 0])


# speedup vs baseline: 2.3845x; 1.0026x over previous
import functools

import jax
import jax.numpy as jnp
from jax.experimental import pallas as pl
from jax.experimental.pallas import tpu as pltpu

B = 8192
D = 256
E = 16
H = 2048
O = 256

BT = 512
NI = B // BT


def _gating_kernel(x_ref, wg_ref, gates_ref, loss_ref):
    lg = jnp.dot(x_ref[...], wg_ref[...], preferred_element_type=jnp.float32)
    m = jnp.max(lg, axis=1, keepdims=True)
    ex = jnp.exp(lg - m)
    g = ex / jnp.sum(ex, axis=1, keepdims=True)
    gates_ref[...] = g
    imp = jnp.sum(g, axis=0)
    load = jnp.sum((g > 0).astype(jnp.float32), axis=0)

    def cv_sq(v):
        mu = jnp.mean(v)
        var = jnp.sum((v - mu) ** 2) / (E - 1)
        return var / (mu * mu + 1e-10)

    loss_ref[...] = jnp.broadcast_to(cv_sq(imp) + cv_sq(load), (1, 1))


def _expert_kernel(x_ref, w1_ref, b1_ref, w2_ref, b2_ref, gates_ref, y_ref):
    e = pl.program_id(0)
    i = pl.program_id(1)
    rows = pl.ds(i * BT, BT)

    xb = x_ref[rows, :]
    h = jnp.dot(xb, w1_ref[0], preferred_element_type=jnp.float32)
    h = jnp.maximum(h.astype(jnp.bfloat16) + b1_ref[0], jnp.bfloat16(0))
    l2 = jnp.dot(h, w2_ref[0], preferred_element_type=jnp.float32)
    l2 = l2 + b2_ref[0]
    p = jnp.exp(l2)
    p = p / jnp.sum(p, axis=1, keepdims=True)

    g_blk = gates_ref[rows, :]
    onehot = (jax.lax.broadcasted_iota(jnp.int32, (1, E), 1) == e)
    gate_col = jnp.sum(g_blk * onehot.astype(jnp.float32), axis=1,
                       keepdims=True)
    contrib = gate_col * p

    @pl.when(e == 0)
    def _():
        y_ref[rows, :] = contrib

    @pl.when(e > 0)
    def _():
        y_ref[rows, :] = y_ref[rows, :] + contrib


@functools.partial(jax.jit, static_argnames=("interpret",))
def kernel(x, w_gate, W1, b1, W2, b2, interpret=False):
    gates, loss = pl.pallas_call(
        _gating_kernel,
        out_shape=[
            jax.ShapeDtypeStruct((B, E), jnp.float32),
            jax.ShapeDtypeStruct((1, 1), jnp.float32),
        ],
        interpret=interpret,
    )(x, w_gate)

    x16 = x.astype(jnp.bfloat16)
    W1_16 = W1.astype(jnp.bfloat16)
    W2_16 = W2.astype(jnp.bfloat16)

    y = pl.pallas_call(
        _expert_kernel,
        grid=(E, NI),
        in_specs=[
            pl.BlockSpec((B, D), lambda e, i: (0, 0)),
            pl.BlockSpec((1, D, H), lambda e, i: (e, 0, 0)),
            pl.BlockSpec((1, 1, H), lambda e, i: (e, 0, 0)),
            pl.BlockSpec((1, H, O), lambda e, i: (e, 0, 0)),
            pl.BlockSpec((1, 1, O), lambda e, i: (e, 0, 0)),
            pl.BlockSpec((B, E), lambda e, i: (0, 0)),
        ],
        out_specs=pl.BlockSpec((B, O), lambda e, i: (0, 0)),
        out_shape=jax.ShapeDtypeStruct((B, O), jnp.float32),
        interpret=interpret,
    )(x16, W1_16, b1.reshape(E, 1, H).astype(jnp.bfloat16), W2_16,
      b2.reshape(E, 1, O), gates)

    return (y, loss[0, 0])


# speedup vs baseline: 2.5379x; 1.0643x over previous
import functools

import jax
import jax.numpy as jnp
from jax.experimental import pallas as pl
from jax.experimental.pallas import tpu as pltpu

B = 8192
D = 256
E = 16
H = 2048
O = 256

BT = 512
NI = B // BT
T_STEPS = E * NI


def _gating_kernel(x_ref, wg_ref, gates_ref, loss_ref):
    lg = jnp.dot(x_ref[...], wg_ref[...], preferred_element_type=jnp.float32)
    m = jnp.max(lg, axis=1, keepdims=True)
    ex = jnp.exp(lg - m)
    g = ex / jnp.sum(ex, axis=1, keepdims=True)
    gates_ref[...] = g
    imp = jnp.sum(g, axis=0)
    load = jnp.sum((g > 0).astype(jnp.float32), axis=0)

    def cv_sq(v):
        mu = jnp.mean(v)
        var = jnp.sum((v - mu) ** 2) / (E - 1)
        return var / (mu * mu + 1e-10)

    loss_ref[...] = jnp.broadcast_to(cv_sq(imp) + cv_sq(load), (1, 1))


def _expert_kernel(x_ref, w1_ref, b1_ref, w2_ref, b2_ref, gates_ref, y_ref,
                   l2_scr):
    t = pl.program_id(0)

    @pl.when(t == 0)
    def _():
        y_ref[...] = jnp.zeros((B, O), jnp.float32)
        l2_scr[...] = jnp.zeros((2, BT, O), jnp.float32)

    tp = jnp.maximum(t - 1, 0)
    ep = tp // NI
    ip = tp % NI
    prows = pl.ds(ip * BT, BT)
    lp = l2_scr[tp % 2]
    p = jnp.exp(lp)
    r = 1.0 / jnp.sum(p, axis=1, keepdims=True)
    g_blk = gates_ref[prows, :]
    onehot = (jax.lax.broadcasted_iota(jnp.int32, (1, E), 1) == ep)
    g_col = jnp.sum(g_blk * onehot.astype(jnp.float32), axis=1,
                    keepdims=True)
    g_col = g_col * (t > 0).astype(jnp.float32)
    y_ref[prows, :] = y_ref[prows, :] + p * (g_col * r)

    tc = jnp.minimum(t, T_STEPS - 1)
    i = tc % NI
    rows = pl.ds(i * BT, BT)
    xb = x_ref[rows, :]
    h = jnp.dot(xb, w1_ref[0], preferred_element_type=jnp.float32)
    h = jnp.maximum(h.astype(jnp.bfloat16) + b1_ref[0], jnp.bfloat16(0))
    l2 = jnp.dot(h, w2_ref[0], preferred_element_type=jnp.float32)
    l2_scr[t % 2] = l2 + b2_ref[0]


@functools.partial(jax.jit, static_argnames=("interpret",))
def kernel(x, w_gate, W1, b1, W2, b2, interpret=False):
    gates, loss = pl.pallas_call(
        _gating_kernel,
        out_shape=[
            jax.ShapeDtypeStruct((B, E), jnp.float32),
            jax.ShapeDtypeStruct((1, 1), jnp.float32),
        ],
        interpret=interpret,
    )(x, w_gate)

    x16 = x.astype(jnp.bfloat16)
    W1_16 = W1.astype(jnp.bfloat16)
    W2_16 = W2.astype(jnp.bfloat16)

    def e_of(t):
        return jnp.minimum(t // NI, E - 1)

    y = pl.pallas_call(
        _expert_kernel,
        grid=(T_STEPS + 1,),
        in_specs=[
            pl.BlockSpec((B, D), lambda t: (0, 0)),
            pl.BlockSpec((1, D, H), lambda t: (e_of(t), 0, 0)),
            pl.BlockSpec((1, 1, H), lambda t: (e_of(t), 0, 0)),
            pl.BlockSpec((1, H, O), lambda t: (e_of(t), 0, 0)),
            pl.BlockSpec((1, 1, O), lambda t: (e_of(t), 0, 0)),
            pl.BlockSpec((B, E), lambda t: (0, 0)),
        ],
        out_specs=pl.BlockSpec((B, O), lambda t: (0, 0)),
        out_shape=jax.ShapeDtypeStruct((B, O), jnp.float32),
        scratch_shapes=[pltpu.VMEM((2, BT, O), jnp.float32)],
        interpret=interpret,
    )(x16, W1_16, b1.reshape(E, 1, H).astype(jnp.bfloat16), W2_16,
      b2.reshape(E, 1, O), gates)

    return (y, loss[0, 0])


# speedup vs baseline: 2.8394x; 1.1188x over previous
import functools

import jax
import jax.numpy as jnp
from jax.experimental import pallas as pl
from jax.experimental.pallas import tpu as pltpu

B = 8192
D = 256
E = 16
H = 2048
O = 256

BT = 1024
NI = B // BT
T_STEPS = E * NI


def _gating_kernel(x_ref, wg_ref, gates_ref, loss_ref):
    lg = jnp.dot(x_ref[...], wg_ref[...], preferred_element_type=jnp.float32)
    m = jnp.max(lg, axis=1, keepdims=True)
    ex = jnp.exp(lg - m)
    g = ex / jnp.sum(ex, axis=1, keepdims=True)
    gates_ref[...] = g
    imp = jnp.sum(g, axis=0)
    load = jnp.sum((g > 0).astype(jnp.float32), axis=0)

    def cv_sq(v):
        mu = jnp.mean(v)
        var = jnp.sum((v - mu) ** 2) / (E - 1)
        return var / (mu * mu + 1e-10)

    loss_ref[...] = jnp.broadcast_to(cv_sq(imp) + cv_sq(load), (1, 1))


def _expert_kernel(x_ref, w1_ref, b1_ref, w2_ref, b2_ref, gates_ref, y_ref,
                   l2_scr):
    t = pl.program_id(0)

    @pl.when(t == 0)
    def _():
        y_ref[...] = jnp.zeros((B, O), jnp.float32)
        l2_scr[...] = jnp.zeros((2, BT, O), jnp.float32)

    tp = jnp.maximum(t - 1, 0)
    ep = tp // NI
    ip = tp % NI
    prows = pl.ds(ip * BT, BT)
    lp = l2_scr[tp % 2]
    p = jnp.exp(lp)
    r = 1.0 / jnp.sum(p, axis=1, keepdims=True)
    g_blk = gates_ref[prows, :]
    onehot = (jax.lax.broadcasted_iota(jnp.int32, (1, E), 1) == ep)
    g_col = jnp.sum(g_blk * onehot.astype(jnp.float32), axis=1,
                    keepdims=True)
    g_col = g_col * (t > 0).astype(jnp.float32)
    y_ref[prows, :] = y_ref[prows, :] + p * (g_col * r)

    tc = jnp.minimum(t, T_STEPS - 1)
    i = tc % NI
    rows = pl.ds(i * BT, BT)
    xb = x_ref[rows, :]
    h = jnp.dot(xb, w1_ref[0], preferred_element_type=jnp.float32)
    h = jnp.maximum(h.astype(jnp.bfloat16) + b1_ref[0], jnp.bfloat16(0))
    l2 = jnp.dot(h, w2_ref[0], preferred_element_type=jnp.float32)
    l2_scr[t % 2] = l2 + b2_ref[0]


@functools.partial(jax.jit, static_argnames=("interpret",))
def kernel(x, w_gate, W1, b1, W2, b2, interpret=False):
    gates, loss = pl.pallas_call(
        _gating_kernel,
        out_shape=[
            jax.ShapeDtypeStruct((B, E), jnp.float32),
            jax.ShapeDtypeStruct((1, 1), jnp.float32),
        ],
        interpret=interpret,
    )(x, w_gate)

    x16 = x.astype(jnp.bfloat16)
    W1_16 = W1.astype(jnp.bfloat16)
    W2_16 = W2.astype(jnp.bfloat16)

    def e_of(t):
        return jnp.minimum(t // NI, E - 1)

    y = pl.pallas_call(
        _expert_kernel,
        grid=(T_STEPS + 1,),
        in_specs=[
            pl.BlockSpec((B, D), lambda t: (0, 0)),
            pl.BlockSpec((1, D, H), lambda t: (e_of(t), 0, 0)),
            pl.BlockSpec((1, 1, H), lambda t: (e_of(t), 0, 0)),
            pl.BlockSpec((1, H, O), lambda t: (e_of(t), 0, 0)),
            pl.BlockSpec((1, 1, O), lambda t: (e_of(t), 0, 0)),
            pl.BlockSpec((B, E), lambda t: (0, 0)),
        ],
        out_specs=pl.BlockSpec((B, O), lambda t: (0, 0)),
        out_shape=jax.ShapeDtypeStruct((B, O), jnp.float32),
        scratch_shapes=[pltpu.VMEM((2, BT, O), jnp.float32)],
        interpret=interpret,
    )(x16, W1_16, b1.reshape(E, 1, H).astype(jnp.bfloat16), W2_16,
      b2.reshape(E, 1, O), gates)

    return (y, loss[0, 0])


# speedup vs baseline: 2.9002x; 1.0214x over previous
import functools

import jax
import jax.numpy as jnp
from jax.experimental import pallas as pl
from jax.experimental.pallas import tpu as pltpu

B = 8192
D = 256
E = 16
H = 2048
O = 256

BT = 2048
NI = B // BT
T_STEPS = E * NI


def _gating_kernel(x_ref, wg_ref, gates_ref, loss_ref):
    lg = jnp.dot(x_ref[...], wg_ref[...], preferred_element_type=jnp.float32)
    m = jnp.max(lg, axis=1, keepdims=True)
    ex = jnp.exp(lg - m)
    g = ex / jnp.sum(ex, axis=1, keepdims=True)
    gates_ref[...] = g
    imp = jnp.sum(g, axis=0)
    load = jnp.sum((g > 0).astype(jnp.float32), axis=0)

    def cv_sq(v):
        mu = jnp.mean(v)
        var = jnp.sum((v - mu) ** 2) / (E - 1)
        return var / (mu * mu + 1e-10)

    loss_ref[...] = jnp.broadcast_to(cv_sq(imp) + cv_sq(load), (1, 1))


def _expert_kernel(x_ref, w1_ref, b1_ref, w2_ref, b2_ref, gates_ref, y_ref,
                   l2_scr):
    t = pl.program_id(0)

    @pl.when(t == 0)
    def _():
        y_ref[...] = jnp.zeros((B, O), jnp.float32)
        l2_scr[...] = jnp.zeros((2, BT, O), jnp.float32)

    tp = jnp.maximum(t - 1, 0)
    ep = tp // NI
    ip = tp % NI
    prows = pl.ds(ip * BT, BT)
    lp = l2_scr[tp % 2]
    p = jnp.exp(lp)
    r = 1.0 / jnp.sum(p, axis=1, keepdims=True)
    g_blk = gates_ref[prows, :]
    onehot = (jax.lax.broadcasted_iota(jnp.int32, (1, E), 1) == ep)
    g_col = jnp.sum(g_blk * onehot.astype(jnp.float32), axis=1,
                    keepdims=True)
    g_col = g_col * (t > 0).astype(jnp.float32)
    y_ref[prows, :] = y_ref[prows, :] + p * (g_col * r)

    tc = jnp.minimum(t, T_STEPS - 1)
    i = tc % NI
    rows = pl.ds(i * BT, BT)
    xb = x_ref[rows, :]
    h = jnp.dot(xb, w1_ref[0], preferred_element_type=jnp.float32)
    h = jnp.maximum(h.astype(jnp.bfloat16) + b1_ref[0], jnp.bfloat16(0))
    l2 = jnp.dot(h, w2_ref[0], preferred_element_type=jnp.float32)
    l2_scr[t % 2] = l2 + b2_ref[0]


@functools.partial(jax.jit, static_argnames=("interpret",))
def kernel(x, w_gate, W1, b1, W2, b2, interpret=False):
    gates, loss = pl.pallas_call(
        _gating_kernel,
        out_shape=[
            jax.ShapeDtypeStruct((B, E), jnp.float32),
            jax.ShapeDtypeStruct((1, 1), jnp.float32),
        ],
        interpret=interpret,
    )(x, w_gate)

    x16 = x.astype(jnp.bfloat16)
    W1_16 = W1.astype(jnp.bfloat16)
    W2_16 = W2.astype(jnp.bfloat16)

    def e_of(t):
        return jnp.minimum(t // NI, E - 1)

    y = pl.pallas_call(
        _expert_kernel,
        grid=(T_STEPS + 1,),
        in_specs=[
            pl.BlockSpec((B, D), lambda t: (0, 0)),
            pl.BlockSpec((1, D, H), lambda t: (e_of(t), 0, 0)),
            pl.BlockSpec((1, 1, H), lambda t: (e_of(t), 0, 0)),
            pl.BlockSpec((1, H, O), lambda t: (e_of(t), 0, 0)),
            pl.BlockSpec((1, 1, O), lambda t: (e_of(t), 0, 0)),
            pl.BlockSpec((B, E), lambda t: (0, 0)),
        ],
        out_specs=pl.BlockSpec((B, O), lambda t: (0, 0)),
        out_shape=jax.ShapeDtypeStruct((B, O), jnp.float32),
        scratch_shapes=[pltpu.VMEM((2, BT, O), jnp.float32)],
        interpret=interpret,
    )(x16, W1_16, b1.reshape(E, 1, H).astype(jnp.bfloat16), W2_16,
      b2.reshape(E, 1, O), gates)

    return (y, loss[0, 0])


# speedup vs baseline: 3.1657x; 1.0915x over previous
import functools

import jax
import jax.numpy as jnp
from jax.experimental import pallas as pl
from jax.experimental.pallas import tpu as pltpu

B = 8192
D = 256
E = 16
H = 2048
O = 256

BT = 2048
NI = B // BT
T_STEPS = E * NI


def _gating_kernel(x_ref, wg_ref, gates_ref, loss_ref, x16_ref):
    xv = x_ref[...]
    x16_ref[...] = xv.astype(jnp.bfloat16)
    lg = jnp.dot(xv, wg_ref[...], preferred_element_type=jnp.float32)
    m = jnp.max(lg, axis=1, keepdims=True)
    ex = jnp.exp(lg - m)
    g = ex / jnp.sum(ex, axis=1, keepdims=True)
    gates_ref[...] = g
    imp = jnp.sum(g, axis=0)
    load = jnp.sum((g > 0).astype(jnp.float32), axis=0)

    def cv_sq(v):
        mu = jnp.mean(v)
        var = jnp.sum((v - mu) ** 2) / (E - 1)
        return var / (mu * mu + 1e-10)

    loss_ref[...] = jnp.broadcast_to(cv_sq(imp) + cv_sq(load), (1, 1))


def _expert_kernel(x_ref, w1_ref, b1_ref, w2_ref, b2_ref, gates_ref, y_ref,
                   l2_scr, w1b_scr, w2b_scr):
    t = pl.program_id(0)

    @pl.when(t == 0)
    def _():
        y_ref[...] = jnp.zeros((B, O), jnp.float32)
        l2_scr[...] = jnp.zeros((2, BT, O), jnp.float32)

    @pl.when(t % NI == 0)
    def _():
        w1b_scr[...] = w1_ref[0].astype(jnp.bfloat16)
        w2b_scr[...] = w2_ref[0].astype(jnp.bfloat16)

    tp = jnp.maximum(t - 1, 0)
    ep = tp // NI
    ip = tp % NI
    prows = pl.ds(ip * BT, BT)
    lp = l2_scr[tp % 2]
    p = jnp.exp(lp)
    r = 1.0 / jnp.sum(p, axis=1, keepdims=True)
    g_blk = gates_ref[prows, :]
    onehot = (jax.lax.broadcasted_iota(jnp.int32, (1, E), 1) == ep)
    g_col = jnp.sum(g_blk * onehot.astype(jnp.float32), axis=1,
                    keepdims=True)
    g_col = g_col * (t > 0).astype(jnp.float32)
    y_ref[prows, :] = y_ref[prows, :] + p * (g_col * r)

    i = jnp.minimum(t, T_STEPS - 1) % NI
    rows = pl.ds(i * BT, BT)
    xb = x_ref[rows, :]
    h = jnp.dot(xb, w1b_scr[...], preferred_element_type=jnp.float32)
    h = jnp.maximum(h.astype(jnp.bfloat16) + b1_ref[0], jnp.bfloat16(0))
    l2 = jnp.dot(h, w2b_scr[...], preferred_element_type=jnp.float32)
    l2_scr[t % 2] = l2 + b2_ref[0]


@functools.partial(jax.jit, static_argnames=("interpret",))
def kernel(x, w_gate, W1, b1, W2, b2, interpret=False):
    gates, loss, x16 = pl.pallas_call(
        _gating_kernel,
        out_shape=[
            jax.ShapeDtypeStruct((B, E), jnp.float32),
            jax.ShapeDtypeStruct((1, 1), jnp.float32),
            jax.ShapeDtypeStruct((B, D), jnp.bfloat16),
        ],
        interpret=interpret,
    )(x, w_gate)

    def e_of(t):
        return jnp.minimum(t // NI, E - 1)

    y = pl.pallas_call(
        _expert_kernel,
        grid=(T_STEPS + 1,),
        in_specs=[
            pl.BlockSpec((B, D), lambda t: (0, 0)),
            pl.BlockSpec((1, D, H), lambda t: (e_of(t), 0, 0)),
            pl.BlockSpec((1, 1, H), lambda t: (e_of(t), 0, 0)),
            pl.BlockSpec((1, H, O), lambda t: (e_of(t), 0, 0)),
            pl.BlockSpec((1, 1, O), lambda t: (e_of(t), 0, 0)),
            pl.BlockSpec((B, E), lambda t: (0, 0)),
        ],
        out_specs=pl.BlockSpec((B, O), lambda t: (0, 0)),
        out_shape=jax.ShapeDtypeStruct((B, O), jnp.float32),
        scratch_shapes=[
            pltpu.VMEM((2, BT, O), jnp.float32),
            pltpu.VMEM((D, H), jnp.bfloat16),
            pltpu.VMEM((H, O), jnp.bfloat16),
        ],
        interpret=interpret,
    )(x16, W1, b1.reshape(E, 1, H).astype(jnp.bfloat16), W2,
      b2.reshape(E, 1, O), gates)

    return (y, loss[0, 0])


# speedup vs baseline: 3.1813x; 1.0049x over previous
import functools

import jax
import jax.numpy as jnp
from jax.experimental import pallas as pl
from jax.experimental.pallas import tpu as pltpu

B = 8192
D = 256
E = 16
H = 2048
O = 256

BT = 4096
NI = B // BT
T_STEPS = E * NI


def _gating_kernel(x_ref, wg_ref, gates_ref, loss_ref, x16_ref):
    xv = x_ref[...]
    x16_ref[...] = xv.astype(jnp.bfloat16)
    lg = jnp.dot(xv, wg_ref[...], preferred_element_type=jnp.float32)
    m = jnp.max(lg, axis=1, keepdims=True)
    ex = jnp.exp(lg - m)
    g = ex / jnp.sum(ex, axis=1, keepdims=True)
    gates_ref[...] = g
    imp = jnp.sum(g, axis=0)
    load = jnp.sum((g > 0).astype(jnp.float32), axis=0)

    def cv_sq(v):
        mu = jnp.mean(v)
        var = jnp.sum((v - mu) ** 2) / (E - 1)
        return var / (mu * mu + 1e-10)

    loss_ref[...] = jnp.broadcast_to(cv_sq(imp) + cv_sq(load), (1, 1))


def _expert_kernel(x_ref, w1_ref, b1_ref, w2_ref, b2_ref, gates_ref, y_ref,
                   l2_scr, w1b_scr, w2b_scr):
    t = pl.program_id(0)

    @pl.when(t == 0)
    def _():
        y_ref[...] = jnp.zeros((B, O), jnp.float32)
        l2_scr[...] = jnp.zeros((2, BT, O), jnp.float32)

    @pl.when(t % NI == 0)
    def _():
        w1b_scr[...] = w1_ref[0].astype(jnp.bfloat16)
        w2b_scr[...] = w2_ref[0].astype(jnp.bfloat16)

    tp = jnp.maximum(t - 1, 0)
    ep = tp // NI
    ip = tp % NI
    prows = pl.ds(ip * BT, BT)
    lp = l2_scr[tp % 2]
    p = jnp.exp(lp)
    r = 1.0 / jnp.sum(p, axis=1, keepdims=True)
    g_blk = gates_ref[prows, :]
    onehot = (jax.lax.broadcasted_iota(jnp.int32, (1, E), 1) == ep)
    g_col = jnp.sum(g_blk * onehot.astype(jnp.float32), axis=1,
                    keepdims=True)
    g_col = g_col * (t > 0).astype(jnp.float32)
    y_ref[prows, :] = y_ref[prows, :] + p * (g_col * r)

    i = jnp.minimum(t, T_STEPS - 1) % NI
    rows = pl.ds(i * BT, BT)
    xb = x_ref[rows, :]
    h = jnp.dot(xb, w1b_scr[...], preferred_element_type=jnp.float32)
    h = jnp.maximum(h.astype(jnp.bfloat16) + b1_ref[0], jnp.bfloat16(0))
    l2 = jnp.dot(h, w2b_scr[...], preferred_element_type=jnp.float32)
    l2_scr[t % 2] = l2 + b2_ref[0]


@functools.partial(jax.jit, static_argnames=("interpret",))
def kernel(x, w_gate, W1, b1, W2, b2, interpret=False):
    gates, loss, x16 = pl.pallas_call(
        _gating_kernel,
        out_shape=[
            jax.ShapeDtypeStruct((B, E), jnp.float32),
            jax.ShapeDtypeStruct((1, 1), jnp.float32),
            jax.ShapeDtypeStruct((B, D), jnp.bfloat16),
        ],
        interpret=interpret,
    )(x, w_gate)

    def e_of(t):
        return jnp.minimum(t // NI, E - 1)

    y = pl.pallas_call(
        _expert_kernel,
        grid=(T_STEPS + 1,),
        in_specs=[
            pl.BlockSpec((B, D), lambda t: (0, 0)),
            pl.BlockSpec((1, D, H), lambda t: (e_of(t), 0, 0)),
            pl.BlockSpec((1, 1, H), lambda t: (e_of(t), 0, 0)),
            pl.BlockSpec((1, H, O), lambda t: (e_of(t), 0, 0)),
            pl.BlockSpec((1, 1, O), lambda t: (e_of(t), 0, 0)),
            pl.BlockSpec((B, E), lambda t: (0, 0)),
        ],
        out_specs=pl.BlockSpec((B, O), lambda t: (0, 0)),
        out_shape=jax.ShapeDtypeStruct((B, O), jnp.float32),
        scratch_shapes=[
            pltpu.VMEM((2, BT, O), jnp.float32),
            pltpu.VMEM((D, H), jnp.bfloat16),
            pltpu.VMEM((H, O), jnp.bfloat16),
        ],
        interpret=interpret,
    )(x16, W1, b1.reshape(E, 1, H).astype(jnp.bfloat16), W2,
      b2.reshape(E, 1, O), gates)

    return (y, loss[0, 0])
